# baseline (speedup 1.0000x reference)
import functools

import jax
import jax.numpy as jnp
from jax import lax
from jax.experimental import pallas as pl
from jax.experimental.pallas import tpu as pltpu
from jax.experimental.pallas import tpu_sc as plsc

NUM_USERS = 25000
NUM_ITEMS = 25000
N = NUM_USERS + NUM_ITEMS
D = 64
E = 800000
BATCH = 4096

HALF = 25000
HALF_PAD = 25008
PAD_OFF = HALF_PAD - HALF
N_PAD = 2 * HALF_PAD
DUMMY = HALF

NS = 16
CHUNK = 112
NB = 4
NCH = NB * (-(-E // (NS * CHUNK * NB)))
EPT = NCH * CHUNK
E_PAD = NS * EPT
NBLK = NS * NCH

ZROWS = HALF_PAD // NS
ZBLK = 256

BPW = BATCH // (2 * NS)

_mesh = plsc.VectorSubcoreMesh(core_axis_name="c", subcore_axis_name="s")

_GDN = lax.GatherDimensionNumbers(
    offset_dims=(), collapsed_slice_dims=(0,), start_index_map=(0,))


def _lane_bcast(v16, j):
    return lax.gather(v16, jnp.full((16, 1), j, jnp.int32), _GDN,
                      slice_sizes=(1,),
                      mode=lax.GatherScatterMode.PROMISE_IN_BOUNDS)


@functools.partial(
    pl.kernel,
    out_type=jax.ShapeDtypeStruct((N_PAD, D), jnp.float32),
    mesh=_mesh,
    compiler_params=pltpu.CompilerParams(use_tc_tiling_on_sc=False,
                                         needs_layout_passes=False),
    scratch_types=[
        [pltpu.VMEM((3, CHUNK), jnp.int32) for _ in range(NB)],
        [pltpu.VMEM((CHUNK,), jnp.int32) for _ in range(NB)],
        [pltpu.VMEM((CHUNK, D), jnp.float32) for _ in range(NB)],
        pltpu.VMEM_SHARED((HALF_PAD, D), jnp.float32),
        pltpu.SemaphoreType.DMA((NB,)),
        pltpu.SemaphoreType.DMA((NB,)),
        pltpu.SemaphoreType.DMA((NB,)),
    ],
)
def _propagate(zeros_hbm, pidx_hbm, tab_hbm, out_hbm,
               pbufs, didxs, gbufs, acc_sh, semi, semg, sems):
    c = lax.axis_index("c")
    s = lax.axis_index("s")
    base_row = c * HALF

    for k in range(ZROWS // ZBLK):
        pltpu.sync_copy(zeros_hbm,
                        acc_sh.at[pl.ds(s * ZROWS + k * ZBLK, ZBLK)])
    rem = ZROWS % ZBLK
    if rem:
        pltpu.sync_copy(zeros_hbm.at[pl.ds(0, rem)],
                        acc_sh.at[pl.ds(s * ZROWS + ZROWS - rem, rem)])
    plsc.subcore_barrier()

    def issue_idx(k, b):
        pltpu.async_copy(pidx_hbm.at[s * NCH + k], pbufs[b], semi.at[b])

    def wait_idx(b):
        pltpu.make_async_copy(pidx_hbm.at[0], pbufs[b], semi.at[b]).wait()

    def issue_gather(b):
        pltpu.async_copy(tab_hbm.at[pbufs[b].at[0]], gbufs[b], semg.at[b])

    def wait_gather(b):
        pltpu.make_async_copy(tab_hbm.at[pbufs[b].at[0]], gbufs[b],
                              semg.at[b]).wait()

    def issue_scatter(b):
        pltpu.async_copy(gbufs[b], acc_sh.at[didxs[b]], sems.at[b], add=True)

    def wait_scatter(b):
        pltpu.make_async_copy(gbufs[b], acc_sh.at[didxs[b]],
                              sems.at[b]).wait()

    def scale(b):
        for g in range(CHUNK // 16):
            sl = pl.ds(g * 16, 16)
            rv = pbufs[b][1, sl]
            lv = rv - base_row
            ok = (lv >= 0) & (lv < HALF)
            didxs[b][sl] = jnp.where(ok, lv, DUMMY)
            v16 = plsc.bitcast(pbufs[b][2, sl], jnp.float32)
            for j in range(16):
                e = g * 16 + j
                bc = _lane_bcast(v16, j)
                for d in range(D // 16):
                    s2 = pl.ds(d * 16, 16)
                    gbufs[b][e, s2] = gbufs[b][e, s2] * bc

    def step(k, b, wait_prev_scatter=True, issue_next_gather=True,
             issue_next_idx=True):
        b2 = (b + 2) % NB
        wait_gather(b)
        if wait_prev_scatter:
            wait_scatter(b2)
        if issue_next_gather:
            wait_idx(b2)
            issue_gather(b2)
        scale(b)
        issue_scatter(b)
        if issue_next_idx:
            issue_idx(k + NB, b)

    for j in range(NB):
        issue_idx(j, j)
    wait_idx(0)
    issue_gather(0)
    wait_idx(1)
    issue_gather(1)
    step(0, 0, wait_prev_scatter=False)
    step(1, 1, wait_prev_scatter=False)
    step(2, 2)
    step(3, 3)

    def quad_body(q, carry):
        kk = NB * q
        for j in range(NB):
            step(kk + j, j)
        return carry

    lax.fori_loop(1, NCH // NB - 1, quad_body, 0)

    step(NCH - 4, 0, issue_next_idx=False)
    step(NCH - 3, 1, issue_next_idx=False)
    step(NCH - 2, 2, issue_next_gather=False, issue_next_idx=False)
    step(NCH - 1, 3, issue_next_gather=False, issue_next_idx=False)
    wait_scatter(2)
    wait_scatter(3)

    plsc.subcore_barrier()
    ob = c * HALF_PAD + s * ZROWS
    pltpu.sync_copy(acc_sh.at[pl.ds(s * ZROWS, ZROWS)],
                    out_hbm.at[pl.ds(ob, ZROWS)])


@functools.partial(
    pl.kernel,
    out_type=jax.ShapeDtypeStruct((BATCH,), jnp.float32),
    mesh=_mesh,
    compiler_params=pltpu.CompilerParams(use_tc_tiling_on_sc=False,
                                         needs_layout_passes=False),
    scratch_types=[
        pltpu.VMEM((BPW,), jnp.int32),
        pltpu.VMEM((BPW,), jnp.int32),
        [pltpu.VMEM((BPW, D), jnp.float32) for _ in range(4)],
        [pltpu.VMEM((BPW, D), jnp.float32) for _ in range(4)],
        pltpu.VMEM((BPW,), jnp.float32),
        pltpu.SemaphoreType.DMA,
    ],
)
def _finalize(users_hbm, items_hbm, t0, t1, t2, t3, gamma_hbm,
              uidx_v, iidx_v, ubufs, ibufs, gout, sem):
    c = lax.axis_index("c")
    s = lax.axis_index("s")
    w = s * 2 + c
    base = pl.multiple_of(w * BPW, BPW)

    pltpu.sync_copy(users_hbm.at[pl.ds(base, BPW)], uidx_v)
    pltpu.sync_copy(items_hbm.at[pl.ds(base, BPW)], iidx_v)

    tabs = (t0, t1, t2, t3)
    cps = []
    for t in range(4):
        cps.append(pltpu.async_copy(tabs[t].at[uidx_v], ubufs[t], sem))
        cps.append(pltpu.async_copy(tabs[t].at[iidx_v], ibufs[t], sem))
    for cp in cps:
        cp.wait()

    iota16 = lax.iota(jnp.int32, 16)

    def group_body(g, carry):
        e16 = iota16 + g * 16
        acc = jnp.zeros((16,), jnp.float32)
        for d in range(D):
            df = jnp.full((16,), d, jnp.int32)
            uv = plsc.load_gather(ubufs[0], [e16, df])
            iv = plsc.load_gather(ibufs[0], [e16, df])
            for t in range(1, 4):
                uv = uv + plsc.load_gather(ubufs[t], [e16, df])
                iv = iv + plsc.load_gather(ibufs[t], [e16, df])
            acc = acc + uv * iv
        plsc.store_scatter(gout, [e16], acc * jnp.float32(0.0625))
        return carry

    lax.fori_loop(0, BPW // 16, group_body, 0)
    pltpu.sync_copy(gout, gamma_hbm.at[pl.ds(base, BPW)])


def kernel(adj_indices, adj_values, users, items, user_emb, item_emb):
    rows = adj_indices[0].astype(jnp.int32)
    cols = adj_indices[1].astype(jnp.int32)

    cols_m = jnp.where(cols >= HALF, cols + PAD_OFF, cols)

    pad = NBLK * CHUNK - E
    cols_m = jnp.concatenate([cols_m, jnp.zeros((pad,), jnp.int32)])
    rows_p = jnp.concatenate([rows, jnp.full((pad,), -1, jnp.int32)])
    vals_i = lax.bitcast_convert_type(adj_values, jnp.int32)
    vals_p = jnp.concatenate([vals_i, jnp.zeros((pad,), jnp.int32)])

    pidx = (jnp.stack([cols_m, rows_p, vals_p])
            .reshape(3, NBLK, CHUNK).transpose(1, 0, 2))

    zpad = jnp.zeros((PAD_OFF, D), jnp.float32)
    tab0 = jnp.concatenate([user_emb, zpad, item_emb, zpad], axis=0)
    zeros = jnp.zeros((ZBLK, D), jnp.float32)

    tab1 = _propagate(zeros, pidx, tab0)
    tab2 = _propagate(zeros, pidx, tab1)
    tab3 = _propagate(zeros, pidx, tab2)

    users32 = users.astype(jnp.int32)
    items32 = items.astype(jnp.int32) + HALF_PAD
    return _finalize(users32, items32, tab0, tab1, tab2, tab3)


# speedup vs baseline: 1.0544x; 1.0544x over previous
import functools

import jax
import jax.numpy as jnp
from jax import lax
from jax.experimental import pallas as pl
from jax.experimental.pallas import tpu as pltpu
from jax.experimental.pallas import tpu_sc as plsc

NUM_USERS = 25000
NUM_ITEMS = 25000
N = NUM_USERS + NUM_ITEMS
D = 64
E = 800000
BATCH = 4096

HALF = 25000
HALF_PAD = 25088
PAD_OFF = HALF_PAD - HALF
N_PAD = 2 * HALF_PAD
DUMMY = HALF

NS = 16
CHUNK = 128
NCH = 2 * (-(-E // (NS * CHUNK * 2)))
EPT = NCH * CHUNK
E_PAD = NS * EPT
NBLK = NS * NCH + 2

ZROWS = HALF_PAD // NS
ZCH = 196

BPW = BATCH // (2 * NS)

_mesh = plsc.VectorSubcoreMesh(core_axis_name="c", subcore_axis_name="s")

_GDN = lax.GatherDimensionNumbers(
    offset_dims=(), collapsed_slice_dims=(0,), start_index_map=(0,))


def _lane_bcast(v16, j):
    return lax.gather(v16, jnp.full((16, 1), j, jnp.int32), _GDN,
                      slice_sizes=(1,),
                      mode=lax.GatherScatterMode.PROMISE_IN_BOUNDS)


@functools.partial(
    pl.kernel,
    out_type=jax.ShapeDtypeStruct((N_PAD, D), jnp.float32),
    mesh=_mesh,
    compiler_params=pltpu.CompilerParams(use_tc_tiling_on_sc=False,
                                         needs_layout_passes=False),
    scratch_types=[
        [pltpu.VMEM((3, CHUNK), jnp.int32) for _ in range(2)],
        [pltpu.VMEM((CHUNK,), jnp.int32) for _ in range(2)],
        [pltpu.VMEM((CHUNK, D), jnp.float32) for _ in range(2)],
        pltpu.VMEM((ZCH, D), jnp.float32),
        pltpu.VMEM_SHARED((HALF_PAD, D), jnp.float32),
        [pltpu.SemaphoreType.DMA for _ in range(2)],
        [pltpu.SemaphoreType.DMA for _ in range(2)],
        [pltpu.SemaphoreType.DMA for _ in range(2)],
    ],
)
def _propagate(pidx_hbm, tab_hbm, out_hbm,
               pbufs, didxs, gbufs, zbuf, acc_sh, semi, semg, sems):
    c = lax.axis_index("c")
    s = lax.axis_index("s")
    base_row = c * HALF

    zero16 = jnp.zeros((16,), jnp.float32)
    for r in range(ZCH):
        for d in range(D // 16):
            zbuf[r, pl.ds(d * 16, 16)] = zero16
    for k in range(ZROWS // ZCH):
        pltpu.sync_copy(zbuf, acc_sh.at[pl.ds(s * ZROWS + k * ZCH, ZCH)])
    plsc.subcore_barrier()

    def issue_idx(k, b):
        pltpu.async_copy(pidx_hbm.at[s * NCH + k], pbufs[b], semi[b])

    def wait_idx(b):
        pltpu.make_async_copy(pidx_hbm.at[0], pbufs[b], semi[b]).wait()

    def issue_gather(b):
        pltpu.async_copy(tab_hbm.at[pbufs[b].at[0]], gbufs[b], semg[b])

    def wait_gather(b):
        pltpu.make_async_copy(tab_hbm.at[pbufs[b].at[0]], gbufs[b],
                              semg[b]).wait()

    def issue_scatter(b):
        pltpu.async_copy(gbufs[b], acc_sh.at[didxs[b]], sems[b], add=True)

    def wait_scatter(b):
        pltpu.make_async_copy(gbufs[b], acc_sh.at[didxs[b]],
                              sems[b]).wait()

    def scale(b):
        for g in range(CHUNK // 16):
            sl = pl.ds(g * 16, 16)
            rv = pbufs[b][1, sl]
            lv = rv - base_row
            ok = (lv >= 0) & (lv < HALF)
            didxs[b][sl] = jnp.where(ok, lv, DUMMY)
            v16 = plsc.bitcast(pbufs[b][2, sl], jnp.float32)
            for j in range(16):
                e = g * 16 + j
                bc = _lane_bcast(v16, j)
                for d in range(D // 16):
                    s2 = pl.ds(d * 16, 16)
                    gbufs[b][e, s2] = gbufs[b][e, s2] * bc

    def step(k, b, wait_prev_scatter=True, issue_next_gather=True,
             issue_next_idx=True):
        o = 1 - b
        wait_gather(b)
        if wait_prev_scatter:
            wait_scatter(o)
        if issue_next_gather:
            wait_idx(o)
            issue_gather(o)
        scale(b)
        issue_scatter(b)
        if issue_next_idx:
            issue_idx(k + 2, b)

    issue_idx(0, 0)
    wait_idx(0)
    issue_gather(0)
    issue_idx(1, 1)
    step(0, 0, wait_prev_scatter=False)

    def pair_body(i, carry):
        kk = 2 * i + 1
        step(kk, 1)
        step(kk + 1, 0)
        return carry

    lax.fori_loop(0, (NCH - 2) // 2, pair_body, 0)

    step(NCH - 1, 1, issue_next_gather=False, issue_next_idx=False)
    wait_idx(0)
    wait_scatter(1)

    plsc.subcore_barrier()
    ob = c * HALF_PAD + s * ZROWS
    pltpu.sync_copy(acc_sh.at[pl.ds(s * ZROWS, ZROWS)],
                    out_hbm.at[pl.ds(ob, ZROWS)])


@functools.partial(
    pl.kernel,
    out_type=jax.ShapeDtypeStruct((BATCH,), jnp.float32),
    mesh=_mesh,
    compiler_params=pltpu.CompilerParams(use_tc_tiling_on_sc=False,
                                         needs_layout_passes=False),
    scratch_types=[
        pltpu.VMEM((BPW,), jnp.int32),
        pltpu.VMEM((BPW,), jnp.int32),
        [pltpu.VMEM((BPW, D), jnp.float32) for _ in range(4)],
        [pltpu.VMEM((BPW, D), jnp.float32) for _ in range(4)],
        pltpu.VMEM((BPW,), jnp.float32),
        pltpu.SemaphoreType.DMA,
    ],
)
def _finalize(users_hbm, items_hbm, t0, t1, t2, t3, gamma_hbm,
              uidx_v, iidx_v, ubufs, ibufs, gout, sem):
    c = lax.axis_index("c")
    s = lax.axis_index("s")
    w = s * 2 + c
    base = pl.multiple_of(w * BPW, BPW)

    pltpu.sync_copy(users_hbm.at[pl.ds(base, BPW)], uidx_v)
    pltpu.sync_copy(items_hbm.at[pl.ds(base, BPW)], iidx_v)

    tabs = (t0, t1, t2, t3)
    cps = []
    for t in range(4):
        cps.append(pltpu.async_copy(tabs[t].at[uidx_v], ubufs[t], sem))
        cps.append(pltpu.async_copy(tabs[t].at[iidx_v], ibufs[t], sem))
    for cp in cps:
        cp.wait()

    iota16 = lax.iota(jnp.int32, 16)

    def group_body(g, carry):
        e16 = iota16 + g * 16
        acc = jnp.zeros((16,), jnp.float32)
        for d in range(D):
            df = jnp.full((16,), d, jnp.int32)
            uv = plsc.load_gather(ubufs[0], [e16, df])
            iv = plsc.load_gather(ibufs[0], [e16, df])
            for t in range(1, 4):
                uv = uv + plsc.load_gather(ubufs[t], [e16, df])
                iv = iv + plsc.load_gather(ibufs[t], [e16, df])
            acc = acc + uv * iv
        plsc.store_scatter(gout, [e16], acc * jnp.float32(0.0625))
        return carry

    lax.fori_loop(0, BPW // 16, group_body, 0)
    pltpu.sync_copy(gout, gamma_hbm.at[pl.ds(base, BPW)])


def kernel(adj_indices, adj_values, users, items, user_emb, item_emb):
    rows = adj_indices[0].astype(jnp.int32)
    cols = adj_indices[1].astype(jnp.int32)

    cols_m = jnp.where(cols >= HALF, cols + PAD_OFF, cols)

    pad = NBLK * CHUNK - E
    cols_m = jnp.concatenate([cols_m, jnp.zeros((pad,), jnp.int32)])
    rows_p = jnp.concatenate([rows, jnp.full((pad,), -1, jnp.int32)])
    vals_i = lax.bitcast_convert_type(adj_values, jnp.int32)
    vals_p = jnp.concatenate([vals_i, jnp.zeros((pad,), jnp.int32)])

    pidx = (jnp.stack([cols_m, rows_p, vals_p])
            .reshape(3, NBLK, CHUNK).transpose(1, 0, 2))

    zpad = jnp.zeros((PAD_OFF, D), jnp.float32)
    tab0 = jnp.concatenate([user_emb, zpad, item_emb, zpad], axis=0)

    tab1 = _propagate(pidx, tab0)
    tab2 = _propagate(pidx, tab1)
    tab3 = _propagate(pidx, tab2)

    users32 = users.astype(jnp.int32)
    items32 = items.astype(jnp.int32) + HALF_PAD
    return _finalize(users32, items32, tab0, tab1, tab2, tab3)
